# baseline (speedup 1.0000x reference)
import jax
import jax.numpy as jnp
from jax import lax
from jax.experimental import pallas as pl
from jax.experimental.pallas import tpu as pltpu
from jax.experimental.pallas import tpu_sc as plsc

N = 100000
D_TOT = 128
D_LAT = 120
HIDDEN = 64

_NW = 32
_SC_ROWS = 400
_GS = 80
_NG = _SC_ROWS // _GS
_NSC = N // _SC_ROWS
_ITERS = -(-_NSC // _NW)

_ROWS_BLK = 2000


def _precompute_body(x_ref, w_ref, ta_ref, tb_ref):
    g = jnp.dot(x_ref[...], w_ref[...], preferred_element_type=jnp.float32)
    gb = g.astype(jnp.bfloat16)
    ta_ref[...] = gb[:, 0:2 * HIDDEN]
    tb_ref[...] = gb[:, HIDDEN:3 * HIDDEN]


def _update_body(x_ref, za_ref, zb_ref, zc_ref, w0_ref, b1_ref, w2_ref,
                 b2_ref, o_ref):
    xb = x_ref[...]
    s = (za_ref[:, 0:HIDDEN].astype(jnp.float32)
         + zb_ref[:, HIDDEN:2 * HIDDEN].astype(jnp.float32)
         + zc_ref[:, HIDDEN:2 * HIDDEN].astype(jnp.float32))
    h = jnp.tanh(
        jnp.dot(xb, w0_ref[...], preferred_element_type=jnp.float32)
        + s + b1_ref[...])
    o_ref[...] = xb + jnp.dot(h, w2_ref[...],
                              preferred_element_type=jnp.float32) + b2_ref[...]


def _sc_body(ta, tb, nT, o1, o2, o3, i_v, a_v, b_v, c_v, sem_i, sem_g, sem_o):
    wid = lax.axis_index("s") * 2 + lax.axis_index("c")

    def issue_idx(it):
        ch = wid + _NW * it

        @pl.when(ch < _NSC)
        def _():
            pltpu.make_async_copy(
                nT.at[:, pl.ds(ch * _SC_ROWS, _SC_ROWS)],
                i_v.at[it % 2], sem_i).start()

    def drain_out():
        pltpu.make_async_copy(a_v, o1.at[pl.ds(0, _SC_ROWS)], sem_o).wait()
        pltpu.make_async_copy(b_v, o2.at[pl.ds(0, _SC_ROWS)], sem_o).wait()
        pltpu.make_async_copy(c_v, o3.at[pl.ds(0, _SC_ROWS)], sem_o).wait()

    issue_idx(0)

    def body(it, carry):
        ch = wid + _NW * it
        p = it % 2

        @pl.when(ch < _NSC)
        def _():
            base = ch * _SC_ROWS
            pltpu.make_async_copy(
                nT.at[:, pl.ds(0, _SC_ROWS)], i_v.at[p], sem_i).wait()
            @pl.when(it > 0)
            def _():
                drain_out()

            cps = []
            for k in range(_NG):
                rs = pl.ds(k * _GS, _GS)
                cps.append(pltpu.async_copy(
                    ta.at[i_v.at[p, 0, rs]], a_v.at[rs], sem_g))
                cps.append(pltpu.async_copy(
                    ta.at[i_v.at[p, 1, rs]], b_v.at[rs], sem_g))
                cps.append(pltpu.async_copy(
                    tb.at[i_v.at[p, 2, rs]], c_v.at[rs], sem_g))
            issue_idx(it + 1)
            for cp in cps:
                cp.wait()

            pltpu.async_copy(a_v, o1.at[pl.ds(base, _SC_ROWS)], sem_o)
            pltpu.async_copy(b_v, o2.at[pl.ds(base, _SC_ROWS)], sem_o)
            pltpu.async_copy(c_v, o3.at[pl.ds(base, _SC_ROWS)], sem_o)

        return carry

    lax.fori_loop(0, _ITERS, body, 0)
    drain_out()


_sc_gather = pl.kernel(
    _sc_body,
    out_type=[jax.ShapeDtypeStruct((N, 2 * HIDDEN), jnp.bfloat16)] * 3,
    mesh=plsc.VectorSubcoreMesh(core_axis_name="c", subcore_axis_name="s"),
    compiler_params=pltpu.CompilerParams(use_tc_tiling_on_sc=False),
    scratch_types=[
        pltpu.VMEM((2, 3, _SC_ROWS), jnp.int32),
        pltpu.VMEM((_SC_ROWS, 2 * HIDDEN), jnp.bfloat16),
        pltpu.VMEM((_SC_ROWS, 2 * HIDDEN), jnp.bfloat16),
        pltpu.VMEM((_SC_ROWS, 2 * HIDDEN), jnp.bfloat16),
        pltpu.SemaphoreType.DMA,
        pltpu.SemaphoreType.DMA,
        pltpu.SemaphoreType.DMA,
    ],
)


def kernel(x, neighbour_index, W1, b1, W2, b2):
    W1r = W1.reshape(4, D_TOT, HIDDEN)
    w1cat = jnp.concatenate([W1r[1], W1r[2], W1r[3]], axis=1)
    w0 = W1r[0]
    w2p = jnp.pad(W2, ((0, 0), (0, D_TOT - D_LAT)))
    b2p = jnp.pad(b2, (0, D_TOT - D_LAT)).reshape(1, D_TOT)
    b1r = b1.reshape(1, HIDDEN)
    nT = neighbour_index.T[1:4]

    grid = (N // _ROWS_BLK,)
    ta, tb = pl.pallas_call(
        _precompute_body,
        grid=grid,
        in_specs=[pl.BlockSpec((_ROWS_BLK, D_TOT), lambda i: (i, 0)),
                  pl.BlockSpec((D_TOT, 3 * HIDDEN), lambda i: (0, 0))],
        out_specs=[pl.BlockSpec((_ROWS_BLK, 2 * HIDDEN), lambda i: (i, 0))] * 2,
        out_shape=[jax.ShapeDtypeStruct((N, 2 * HIDDEN), jnp.bfloat16)] * 2,
    )(x, w1cat)

    za, zb, zc = _sc_gather(ta, tb, nT)

    out = pl.pallas_call(
        _update_body,
        grid=grid,
        in_specs=[pl.BlockSpec((_ROWS_BLK, D_TOT), lambda i: (i, 0)),
                  pl.BlockSpec((_ROWS_BLK, 2 * HIDDEN), lambda i: (i, 0)),
                  pl.BlockSpec((_ROWS_BLK, 2 * HIDDEN), lambda i: (i, 0)),
                  pl.BlockSpec((_ROWS_BLK, 2 * HIDDEN), lambda i: (i, 0)),
                  pl.BlockSpec((D_TOT, HIDDEN), lambda i: (0, 0)),
                  pl.BlockSpec((1, HIDDEN), lambda i: (0, 0)),
                  pl.BlockSpec((HIDDEN, D_TOT), lambda i: (0, 0)),
                  pl.BlockSpec((1, D_TOT), lambda i: (0, 0))],
        out_specs=pl.BlockSpec((_ROWS_BLK, D_TOT), lambda i: (i, 0)),
        out_shape=jax.ShapeDtypeStruct((N, D_TOT), jnp.float32),
    )(x, za, zb, zc, w0, b1r, w2p, b2p)
    return out


# speedup vs baseline: 2.6837x; 2.6837x over previous
import jax
import jax.numpy as jnp
from jax import lax
from jax.experimental import pallas as pl
from jax.experimental.pallas import tpu as pltpu
from jax.experimental.pallas import tpu_sc as plsc

N = 100000
D_TOT = 128
D_LAT = 120
HIDDEN = 64

_NW = 32
_SC_ROWS = 200
_GS = 40
_NG = _SC_ROWS // _GS
_NSC = N // _SC_ROWS
_ITERS = -(-_NSC // _NW)

_ROWS_BLK = 2000


def _precompute_body(x_ref, w_ref, ta_ref, tb_ref):
    g = jnp.dot(x_ref[...], w_ref[...], preferred_element_type=jnp.float32)
    ta_ref[...] = g[:, 0:2 * HIDDEN]
    tb_ref[...] = g[:, HIDDEN:3 * HIDDEN]


def _update_body(x_ref, s_ref, w0_ref, b1_ref, w2_ref, b2_ref, o_ref):
    xb = x_ref[...]
    h = jnp.tanh(
        jnp.dot(xb, w0_ref[...], preferred_element_type=jnp.float32)
        + s_ref[:, 0:HIDDEN] + b1_ref[...])
    o_ref[...] = xb + jnp.dot(h, w2_ref[...],
                              preferred_element_type=jnp.float32) + b2_ref[...]


def _sc_body(ta, tb, nT, out, i_v, a_v, b_v, c_v, sem_i, sem_g, sem_o):
    wid = lax.axis_index("s") * 2 + lax.axis_index("c")

    def issue_idx(it):
        ch = wid + _NW * it

        @pl.when(ch < _NSC)
        def _():
            pltpu.make_async_copy(
                nT.at[:, pl.ds(ch * _SC_ROWS, _SC_ROWS)],
                i_v.at[it % 2], sem_i).start()

    issue_idx(0)

    def body(it, carry):
        ch = wid + _NW * it
        p = it % 2

        @pl.when(ch < _NSC)
        def _():
            base = ch * _SC_ROWS
            pltpu.make_async_copy(
                nT.at[:, pl.ds(0, _SC_ROWS)], i_v.at[p], sem_i).wait()
            @pl.when(it > 0)
            def _():
                pltpu.make_async_copy(
                    a_v, out.at[pl.ds(0, _SC_ROWS)], sem_o).wait()

            cps = []
            for k in range(_NG):
                rs = pl.ds(k * _GS, _GS)
                cps.append(pltpu.async_copy(
                    ta.at[i_v.at[p, 0, rs]], a_v.at[rs], sem_g))
                cps.append(pltpu.async_copy(
                    ta.at[i_v.at[p, 1, rs]], b_v.at[rs], sem_g))
                cps.append(pltpu.async_copy(
                    tb.at[i_v.at[p, 2, rs]], c_v.at[rs], sem_g))
            issue_idx(it + 1)
            for cp in cps:
                cp.wait()

            def add_rows(i, c2):
                r = i * 4
                for dr in range(4):
                    for j in range(HIDDEN // 16):
                        sl = pl.ds(j * 16, 16)
                        sh = pl.ds(HIDDEN + j * 16, 16)
                        plsc.addupdate(a_v.at[r + dr, sl],
                                       b_v[r + dr, sh] + c_v[r + dr, sh])
                return c2

            lax.fori_loop(0, _SC_ROWS // 4, add_rows, 0)
            pltpu.async_copy(a_v, out.at[pl.ds(base, _SC_ROWS)], sem_o)

        return carry

    lax.fori_loop(0, _ITERS, body, 0)
    pltpu.make_async_copy(a_v, out.at[pl.ds(0, _SC_ROWS)], sem_o).wait()


_sc_gather_sum = pl.kernel(
    _sc_body,
    out_type=jax.ShapeDtypeStruct((N, 2 * HIDDEN), jnp.float32),
    mesh=plsc.VectorSubcoreMesh(core_axis_name="c", subcore_axis_name="s"),
    compiler_params=pltpu.CompilerParams(use_tc_tiling_on_sc=False),
    scratch_types=[
        pltpu.VMEM((2, 3, _SC_ROWS), jnp.int32),
        pltpu.VMEM((_SC_ROWS, 2 * HIDDEN), jnp.float32),
        pltpu.VMEM((_SC_ROWS, 2 * HIDDEN), jnp.float32),
        pltpu.VMEM((_SC_ROWS, 2 * HIDDEN), jnp.float32),
        pltpu.SemaphoreType.DMA,
        pltpu.SemaphoreType.DMA,
        pltpu.SemaphoreType.DMA,
    ],
)


def kernel(x, neighbour_index, W1, b1, W2, b2):
    W1r = W1.reshape(4, D_TOT, HIDDEN)
    w1cat = jnp.concatenate([W1r[1], W1r[2], W1r[3]], axis=1)
    w0 = W1r[0]
    w2p = jnp.pad(W2, ((0, 0), (0, D_TOT - D_LAT)))
    b2p = jnp.pad(b2, (0, D_TOT - D_LAT)).reshape(1, D_TOT)
    b1r = b1.reshape(1, HIDDEN)
    nT = neighbour_index.T[1:4]

    grid = (N // _ROWS_BLK,)
    ta, tb = pl.pallas_call(
        _precompute_body,
        grid=grid,
        in_specs=[pl.BlockSpec((_ROWS_BLK, D_TOT), lambda i: (i, 0)),
                  pl.BlockSpec((D_TOT, 3 * HIDDEN), lambda i: (0, 0))],
        out_specs=[pl.BlockSpec((_ROWS_BLK, 2 * HIDDEN), lambda i: (i, 0))] * 2,
        out_shape=[jax.ShapeDtypeStruct((N, 2 * HIDDEN), jnp.float32)] * 2,
    )(x, w1cat)

    s = _sc_gather_sum(ta, tb, nT)

    out = pl.pallas_call(
        _update_body,
        grid=grid,
        in_specs=[pl.BlockSpec((_ROWS_BLK, D_TOT), lambda i: (i, 0)),
                  pl.BlockSpec((_ROWS_BLK, 2 * HIDDEN), lambda i: (i, 0)),
                  pl.BlockSpec((D_TOT, HIDDEN), lambda i: (0, 0)),
                  pl.BlockSpec((1, HIDDEN), lambda i: (0, 0)),
                  pl.BlockSpec((HIDDEN, D_TOT), lambda i: (0, 0)),
                  pl.BlockSpec((1, D_TOT), lambda i: (0, 0))],
        out_specs=pl.BlockSpec((_ROWS_BLK, D_TOT), lambda i: (i, 0)),
        out_shape=jax.ShapeDtypeStruct((N, D_TOT), jnp.float32),
    )(x, s, w0, b1r, w2p, b2p)
    return out


# speedup vs baseline: 2.8394x; 1.0580x over previous
import jax
import jax.numpy as jnp
from jax import lax
from jax.experimental import pallas as pl
from jax.experimental.pallas import tpu as pltpu
from jax.experimental.pallas import tpu_sc as plsc

N = 100000
D_TOT = 128
D_LAT = 120
HIDDEN = 64

_NW = 32
_SC_ROWS = 200
_GS = 40
_NG = _SC_ROWS // _GS
_NSC = N // _SC_ROWS
_ITERS = -(-_NSC // _NW)

_ROWS_BLK = 2000


def _precompute_body(x_ref, w_ref, ta_ref, tb_ref):
    g = jnp.dot(x_ref[...], w_ref[...], preferred_element_type=jnp.float32)
    ta_ref[...] = g[:, 0:2 * HIDDEN]
    tb_ref[...] = g[:, HIDDEN:3 * HIDDEN]


def _update_body(x_ref, s_ref, w0_ref, b1_ref, w2_ref, b2_ref, o_ref):
    xb = x_ref[...]
    h = jnp.tanh(
        jnp.dot(xb, w0_ref[...], preferred_element_type=jnp.float32)
        + s_ref[:, 0:HIDDEN] + b1_ref[...])
    o_ref[...] = xb + jnp.dot(h, w2_ref[...],
                              preferred_element_type=jnp.float32) + b2_ref[...]


def _update_body_hi(buf_ref, x_ref, s_ref, w0_ref, b1_ref, w2_ref, b2_ref,
                    o_ref):
    del buf_ref
    _update_body(x_ref, s_ref, w0_ref, b1_ref, w2_ref, b2_ref, o_ref)


_NHALF = N // 2
_NSC_H = _NHALF // _SC_ROWS
_ITERS_H = -(-_NSC_H // _NW)


def _make_sc_body(row_off):
    def _sc_body(ta, tb, nT, out, i_v, a_v, b_v, c_v, sem_i, sem_g, sem_o):
        wid = lax.axis_index("s") * 2 + lax.axis_index("c")

        def issue_idx(it):
            ch = wid + _NW * it

            @pl.when(ch < _NSC_H)
            def _():
                pltpu.make_async_copy(
                    nT.at[:, pl.ds(row_off + ch * _SC_ROWS, _SC_ROWS)],
                    i_v.at[it % 2], sem_i).start()

        issue_idx(0)

        def body(it, carry):
            ch = wid + _NW * it
            p = it % 2

            @pl.when(ch < _NSC_H)
            def _():
                base = ch * _SC_ROWS
                pltpu.make_async_copy(
                    nT.at[:, pl.ds(0, _SC_ROWS)], i_v.at[p], sem_i).wait()
                @pl.when(it > 0)
                def _():
                    pltpu.make_async_copy(
                        a_v, out.at[pl.ds(0, _SC_ROWS)], sem_o).wait()

                cps = []
                for k in range(_NG):
                    rs = pl.ds(k * _GS, _GS)
                    cps.append(pltpu.async_copy(
                        ta.at[i_v.at[p, 0, rs]], a_v.at[rs], sem_g))
                    cps.append(pltpu.async_copy(
                        ta.at[i_v.at[p, 1, rs]], b_v.at[rs], sem_g))
                    cps.append(pltpu.async_copy(
                        tb.at[i_v.at[p, 2, rs]], c_v.at[rs], sem_g))
                issue_idx(it + 1)
                for cp in cps:
                    cp.wait()

                def add_rows(i, c2):
                    r = i * 4
                    for dr in range(4):
                        for j in range(HIDDEN // 16):
                            sl = pl.ds(j * 16, 16)
                            sh = pl.ds(HIDDEN + j * 16, 16)
                            plsc.addupdate(a_v.at[r + dr, sl],
                                           b_v[r + dr, sh] + c_v[r + dr, sh])
                    return c2

                lax.fori_loop(0, _SC_ROWS // 4, add_rows, 0)
                pltpu.async_copy(a_v, out.at[pl.ds(base, _SC_ROWS)], sem_o)

            return carry

        lax.fori_loop(0, _ITERS_H, body, 0)
        pltpu.make_async_copy(a_v, out.at[pl.ds(0, _SC_ROWS)], sem_o).wait()

    return _sc_body


def _make_sc_call(row_off):
    return pl.kernel(
        _make_sc_body(row_off),
        out_type=jax.ShapeDtypeStruct((_NHALF, 2 * HIDDEN), jnp.float32),
        mesh=plsc.VectorSubcoreMesh(core_axis_name="c", subcore_axis_name="s"),
        compiler_params=pltpu.CompilerParams(use_tc_tiling_on_sc=False),
        scratch_types=[
            pltpu.VMEM((2, 3, _SC_ROWS), jnp.int32),
            pltpu.VMEM((_SC_ROWS, 2 * HIDDEN), jnp.float32),
            pltpu.VMEM((_SC_ROWS, 2 * HIDDEN), jnp.float32),
            pltpu.VMEM((_SC_ROWS, 2 * HIDDEN), jnp.float32),
            pltpu.SemaphoreType.DMA,
            pltpu.SemaphoreType.DMA,
            pltpu.SemaphoreType.DMA,
        ],
    )


_sc_gather_lo = _make_sc_call(0)
_sc_gather_hi = _make_sc_call(_NHALF)


def kernel(x, neighbour_index, W1, b1, W2, b2):
    W1r = W1.reshape(4, D_TOT, HIDDEN)
    w1cat = jnp.concatenate([W1r[1], W1r[2], W1r[3]], axis=1)
    w0 = W1r[0]
    w2p = jnp.pad(W2, ((0, 0), (0, D_TOT - D_LAT)))
    b2p = jnp.pad(b2, (0, D_TOT - D_LAT)).reshape(1, D_TOT)
    b1r = b1.reshape(1, HIDDEN)
    nT = neighbour_index.T[1:4]

    grid = (N // _ROWS_BLK,)
    ta, tb = pl.pallas_call(
        _precompute_body,
        grid=grid,
        in_specs=[pl.BlockSpec((_ROWS_BLK, D_TOT), lambda i: (i, 0)),
                  pl.BlockSpec((D_TOT, 3 * HIDDEN), lambda i: (0, 0))],
        out_specs=[pl.BlockSpec((_ROWS_BLK, 2 * HIDDEN), lambda i: (i, 0))] * 2,
        out_shape=[jax.ShapeDtypeStruct((N, 2 * HIDDEN), jnp.float32)] * 2,
    )(x, w1cat)

    s_lo = _sc_gather_lo(ta, tb, nT)
    s_hi = _sc_gather_hi(ta, tb, nT)

    grid_h = (_NHALF // _ROWS_BLK,)
    wspecs = [pl.BlockSpec((D_TOT, HIDDEN), lambda i: (0, 0)),
              pl.BlockSpec((1, HIDDEN), lambda i: (0, 0)),
              pl.BlockSpec((HIDDEN, D_TOT), lambda i: (0, 0)),
              pl.BlockSpec((1, D_TOT), lambda i: (0, 0))]
    out_lo = pl.pallas_call(
        _update_body,
        grid=grid_h,
        in_specs=[pl.BlockSpec((_ROWS_BLK, D_TOT), lambda i: (i, 0)),
                  pl.BlockSpec((_ROWS_BLK, 2 * HIDDEN), lambda i: (i, 0))]
                 + wspecs,
        out_specs=pl.BlockSpec((_ROWS_BLK, D_TOT), lambda i: (i, 0)),
        out_shape=jax.ShapeDtypeStruct((N, D_TOT), jnp.float32),
    )(x, s_lo, w0, b1r, w2p, b2p)

    half_blk = _NHALF // _ROWS_BLK
    out = pl.pallas_call(
        _update_body_hi,
        grid=grid_h,
        in_specs=[pl.BlockSpec((8, D_TOT), lambda i: (0, 0)),
                  pl.BlockSpec((_ROWS_BLK, D_TOT),
                               lambda i: (i + half_blk, 0)),
                  pl.BlockSpec((_ROWS_BLK, 2 * HIDDEN), lambda i: (i, 0))]
                 + wspecs,
        out_specs=pl.BlockSpec((_ROWS_BLK, D_TOT),
                               lambda i: (i + half_blk, 0)),
        out_shape=jax.ShapeDtypeStruct((N, D_TOT), jnp.float32),
        input_output_aliases={0: 0},
    )(out_lo, x, s_hi, w0, b1r, w2p, b2p)
    return out


# speedup vs baseline: 2.9884x; 1.0525x over previous
import jax
import jax.numpy as jnp
from jax import lax
from jax.experimental import pallas as pl
from jax.experimental.pallas import tpu as pltpu
from jax.experimental.pallas import tpu_sc as plsc

N = 100000
D_TOT = 128
D_LAT = 120
HIDDEN = 64

_NW = 32
_SC_ROWS = 200
_GS = 40
_NG = _SC_ROWS // _GS
_NSC = N // _SC_ROWS
_ITERS = -(-_NSC // _NW)

_BLK_PRE = 4000
_ROWS_BLK = 2000


def _precompute_body(x_ref, w_ref, ta_ref, tb_ref):
    g = jnp.dot(x_ref[...], w_ref[...], preferred_element_type=jnp.float32)
    ta_ref[...] = g[:, 0:2 * HIDDEN]
    tb_ref[...] = g[:, HIDDEN:3 * HIDDEN]


def _update_body(x_ref, s_ref, w0_ref, b1_ref, w2_ref, b2_ref, o_ref):
    xb = x_ref[...]
    h = jnp.tanh(
        jnp.dot(xb, w0_ref[...], preferred_element_type=jnp.float32)
        + s_ref[:, 0:HIDDEN] + b1_ref[...])
    o_ref[...] = xb + jnp.dot(h, w2_ref[...],
                              preferred_element_type=jnp.float32) + b2_ref[...]


def _update_body_hi(buf_ref, x_ref, s_ref, w0_ref, b1_ref, w2_ref, b2_ref,
                    o_ref):
    del buf_ref
    _update_body(x_ref, s_ref, w0_ref, b1_ref, w2_ref, b2_ref, o_ref)


_NLO = 60000
_NHI = N - _NLO


def _make_sc_body(row_off, nrows):
    nsc = nrows // _SC_ROWS
    iters = -(-nsc // _NW)

    def _sc_body(ta, tb, nT, out, i_v, a_v, b_v, c_v, sem_i, sem_g, sem_o):
        wid = lax.axis_index("s") * 2 + lax.axis_index("c")

        def issue_idx(it):
            ch = wid + _NW * it

            @pl.when(ch < nsc)
            def _():
                pltpu.make_async_copy(
                    nT.at[:, pl.ds(row_off + ch * _SC_ROWS, _SC_ROWS)],
                    i_v.at[it % 2], sem_i).start()

        issue_idx(0)

        def body(it, carry):
            ch = wid + _NW * it
            p = it % 2

            @pl.when(ch < nsc)
            def _():
                base = ch * _SC_ROWS
                pltpu.make_async_copy(
                    nT.at[:, pl.ds(0, _SC_ROWS)], i_v.at[p], sem_i).wait()
                @pl.when(it > 0)
                def _():
                    pltpu.make_async_copy(
                        a_v, out.at[pl.ds(0, _SC_ROWS)], sem_o).wait()

                cps = []
                for k in range(_NG):
                    rs = pl.ds(k * _GS, _GS)
                    cps.append(pltpu.async_copy(
                        ta.at[i_v.at[p, 0, rs]], a_v.at[rs], sem_g))
                    cps.append(pltpu.async_copy(
                        ta.at[i_v.at[p, 1, rs]], b_v.at[rs], sem_g))
                    cps.append(pltpu.async_copy(
                        tb.at[i_v.at[p, 2, rs]], c_v.at[rs], sem_g))
                issue_idx(it + 1)
                for cp in cps:
                    cp.wait()

                def add_rows(i, c2):
                    r = i * 4
                    for dr in range(4):
                        for j in range(HIDDEN // 16):
                            sl = pl.ds(j * 16, 16)
                            sh = pl.ds(HIDDEN + j * 16, 16)
                            plsc.addupdate(a_v.at[r + dr, sl],
                                           b_v[r + dr, sh] + c_v[r + dr, sh])
                    return c2

                lax.fori_loop(0, _SC_ROWS // 4, add_rows, 0)
                pltpu.async_copy(a_v, out.at[pl.ds(base, _SC_ROWS)], sem_o)

            return carry

        lax.fori_loop(0, iters, body, 0)
        pltpu.make_async_copy(a_v, out.at[pl.ds(0, _SC_ROWS)], sem_o).wait()

    return _sc_body


def _make_sc_call(row_off, nrows):
    return pl.kernel(
        _make_sc_body(row_off, nrows),
        out_type=jax.ShapeDtypeStruct((nrows, 2 * HIDDEN), jnp.float32),
        mesh=plsc.VectorSubcoreMesh(core_axis_name="c", subcore_axis_name="s"),
        compiler_params=pltpu.CompilerParams(use_tc_tiling_on_sc=False),
        scratch_types=[
            pltpu.VMEM((2, 3, _SC_ROWS), jnp.int32),
            pltpu.VMEM((_SC_ROWS, 2 * HIDDEN), jnp.float32),
            pltpu.VMEM((_SC_ROWS, 2 * HIDDEN), jnp.float32),
            pltpu.VMEM((_SC_ROWS, 2 * HIDDEN), jnp.float32),
            pltpu.SemaphoreType.DMA,
            pltpu.SemaphoreType.DMA,
            pltpu.SemaphoreType.DMA,
        ],
    )


_sc_gather_lo = _make_sc_call(0, _NLO)
_sc_gather_hi = _make_sc_call(_NLO, _NHI)


def kernel(x, neighbour_index, W1, b1, W2, b2):
    W1r = W1.reshape(4, D_TOT, HIDDEN)
    w1cat = jnp.concatenate([W1r[1], W1r[2], W1r[3]], axis=1)
    w0 = W1r[0]
    w2p = jnp.pad(W2, ((0, 0), (0, D_TOT - D_LAT)))
    b2p = jnp.pad(b2, (0, D_TOT - D_LAT)).reshape(1, D_TOT)
    b1r = b1.reshape(1, HIDDEN)
    nT = neighbour_index.T[1:4]

    ta, tb = pl.pallas_call(
        _precompute_body,
        grid=(N // _BLK_PRE,),
        in_specs=[pl.BlockSpec((_BLK_PRE, D_TOT), lambda i: (i, 0)),
                  pl.BlockSpec((D_TOT, 3 * HIDDEN), lambda i: (0, 0))],
        out_specs=[pl.BlockSpec((_BLK_PRE, 2 * HIDDEN), lambda i: (i, 0))] * 2,
        out_shape=[jax.ShapeDtypeStruct((N, 2 * HIDDEN), jnp.float32)] * 2,
    )(x, w1cat)

    s_lo = _sc_gather_lo(ta, tb, nT)
    s_hi = _sc_gather_hi(ta, tb, nT)

    wspecs = [pl.BlockSpec((D_TOT, HIDDEN), lambda i: (0, 0)),
              pl.BlockSpec((1, HIDDEN), lambda i: (0, 0)),
              pl.BlockSpec((HIDDEN, D_TOT), lambda i: (0, 0)),
              pl.BlockSpec((1, D_TOT), lambda i: (0, 0))]
    out_lo = pl.pallas_call(
        _update_body,
        grid=(_NLO // _ROWS_BLK,),
        in_specs=[pl.BlockSpec((_ROWS_BLK, D_TOT), lambda i: (i, 0)),
                  pl.BlockSpec((_ROWS_BLK, 2 * HIDDEN), lambda i: (i, 0))]
                 + wspecs,
        out_specs=pl.BlockSpec((_ROWS_BLK, D_TOT), lambda i: (i, 0)),
        out_shape=jax.ShapeDtypeStruct((N, D_TOT), jnp.float32),
    )(x, s_lo, w0, b1r, w2p, b2p)

    lo_blk = _NLO // _ROWS_BLK
    out = pl.pallas_call(
        _update_body_hi,
        grid=(_NHI // _ROWS_BLK,),
        in_specs=[pl.BlockSpec((8, D_TOT), lambda i: (0, 0)),
                  pl.BlockSpec((_ROWS_BLK, D_TOT),
                               lambda i: (i + lo_blk, 0)),
                  pl.BlockSpec((_ROWS_BLK, 2 * HIDDEN), lambda i: (i, 0))]
                 + wspecs,
        out_specs=pl.BlockSpec((_ROWS_BLK, D_TOT),
                               lambda i: (i + lo_blk, 0)),
        out_shape=jax.ShapeDtypeStruct((N, D_TOT), jnp.float32),
        input_output_aliases={0: 0},
    )(out_lo, x, s_hi, w0, b1r, w2p, b2p)
    return out


# speedup vs baseline: 3.0644x; 1.0254x over previous
import jax
import jax.numpy as jnp
from jax import lax
from jax.experimental import pallas as pl
from jax.experimental.pallas import tpu as pltpu
from jax.experimental.pallas import tpu_sc as plsc

N = 100000
D_TOT = 128
D_LAT = 120
HIDDEN = 64

_NW = 32
_SC_ROWS = 200
_GS = 40
_NG = _SC_ROWS // _GS
_NSC = N // _SC_ROWS
_ITERS = -(-_NSC // _NW)

_BLK_PRE = 4000
_ROWS_BLK = 2000


def _precompute_body(x_ref, w_ref, ta_ref, tb_ref):
    g = jnp.dot(x_ref[...], w_ref[...], preferred_element_type=jnp.float32)
    ta_ref[...] = g[:, 0:2 * HIDDEN]
    tb_ref[...] = g[:, HIDDEN:3 * HIDDEN]


def _update_body(x_ref, s_ref, w0_ref, b1_ref, w2_ref, b2_ref, o_ref):
    xb = x_ref[...]
    h = jnp.tanh(
        jnp.dot(xb, w0_ref[...], preferred_element_type=jnp.float32)
        + s_ref[:, 0:HIDDEN] + b1_ref[...])
    o_ref[...] = xb + jnp.dot(h, w2_ref[...],
                              preferred_element_type=jnp.float32) + b2_ref[...]


def _update_body_hi(buf_ref, x_ref, s_ref, w0_ref, b1_ref, w2_ref, b2_ref,
                    o_ref):
    del buf_ref
    _update_body(x_ref, s_ref, w0_ref, b1_ref, w2_ref, b2_ref, o_ref)


_SPLITS = ((0, 56000), (56000, 28000), (84000, 16000))


def _make_sc_body(row_off, nrows):
    nsc = nrows // _SC_ROWS
    iters = -(-nsc // _NW)

    def _sc_body(ta, tb, nT, out, i_v, a_v, b_v, c_v, sem_i, sem_g, sem_o):
        wid = lax.axis_index("s") * 2 + lax.axis_index("c")

        def issue_idx(it):
            ch = wid + _NW * it

            @pl.when(ch < nsc)
            def _():
                pltpu.make_async_copy(
                    nT.at[:, pl.ds(row_off + ch * _SC_ROWS, _SC_ROWS)],
                    i_v.at[it % 2], sem_i).start()

        issue_idx(0)

        def body(it, carry):
            ch = wid + _NW * it
            p = it % 2

            @pl.when(ch < nsc)
            def _():
                base = ch * _SC_ROWS
                pltpu.make_async_copy(
                    nT.at[:, pl.ds(0, _SC_ROWS)], i_v.at[p], sem_i).wait()
                @pl.when(it > 0)
                def _():
                    pltpu.make_async_copy(
                        a_v, out.at[pl.ds(0, _SC_ROWS)], sem_o).wait()

                cps = []
                for k in range(_NG):
                    rs = pl.ds(k * _GS, _GS)
                    cps.append(pltpu.async_copy(
                        ta.at[i_v.at[p, 0, rs]], a_v.at[rs], sem_g))
                    cps.append(pltpu.async_copy(
                        ta.at[i_v.at[p, 1, rs]], b_v.at[rs], sem_g))
                    cps.append(pltpu.async_copy(
                        tb.at[i_v.at[p, 2, rs]], c_v.at[rs], sem_g))
                issue_idx(it + 1)
                for cp in cps:
                    cp.wait()

                def add_rows(i, c2):
                    r = i * 4
                    for dr in range(4):
                        for j in range(HIDDEN // 16):
                            sl = pl.ds(j * 16, 16)
                            sh = pl.ds(HIDDEN + j * 16, 16)
                            plsc.addupdate(a_v.at[r + dr, sl],
                                           b_v[r + dr, sh] + c_v[r + dr, sh])
                    return c2

                lax.fori_loop(0, _SC_ROWS // 4, add_rows, 0)
                pltpu.async_copy(a_v, out.at[pl.ds(base, _SC_ROWS)], sem_o)

            return carry

        lax.fori_loop(0, iters, body, 0)
        pltpu.make_async_copy(a_v, out.at[pl.ds(0, _SC_ROWS)], sem_o).wait()

    return _sc_body


def _make_sc_call(row_off, nrows):
    return pl.kernel(
        _make_sc_body(row_off, nrows),
        out_type=jax.ShapeDtypeStruct((nrows, 2 * HIDDEN), jnp.float32),
        mesh=plsc.VectorSubcoreMesh(core_axis_name="c", subcore_axis_name="s"),
        compiler_params=pltpu.CompilerParams(use_tc_tiling_on_sc=False),
        scratch_types=[
            pltpu.VMEM((2, 3, _SC_ROWS), jnp.int32),
            pltpu.VMEM((_SC_ROWS, 2 * HIDDEN), jnp.float32),
            pltpu.VMEM((_SC_ROWS, 2 * HIDDEN), jnp.float32),
            pltpu.VMEM((_SC_ROWS, 2 * HIDDEN), jnp.float32),
            pltpu.SemaphoreType.DMA,
            pltpu.SemaphoreType.DMA,
            pltpu.SemaphoreType.DMA,
        ],
    )


_sc_gathers = tuple(_make_sc_call(off, n) for off, n in _SPLITS)


def kernel(x, neighbour_index, W1, b1, W2, b2):
    W1r = W1.reshape(4, D_TOT, HIDDEN)
    w1cat = jnp.concatenate([W1r[1], W1r[2], W1r[3]], axis=1)
    w0 = W1r[0]
    w2p = jnp.pad(W2, ((0, 0), (0, D_TOT - D_LAT)))
    b2p = jnp.pad(b2, (0, D_TOT - D_LAT)).reshape(1, D_TOT)
    b1r = b1.reshape(1, HIDDEN)
    nT = neighbour_index.T[1:4]

    ta, tb = pl.pallas_call(
        _precompute_body,
        grid=(N // _BLK_PRE,),
        in_specs=[pl.BlockSpec((_BLK_PRE, D_TOT), lambda i: (i, 0)),
                  pl.BlockSpec((D_TOT, 3 * HIDDEN), lambda i: (0, 0))],
        out_specs=[pl.BlockSpec((_BLK_PRE, 2 * HIDDEN), lambda i: (i, 0))] * 2,
        out_shape=[jax.ShapeDtypeStruct((N, 2 * HIDDEN), jnp.float32)] * 2,
    )(x, w1cat)

    s_parts = [g(ta, tb, nT) for g in _sc_gathers]

    wspecs = [pl.BlockSpec((D_TOT, HIDDEN), lambda i: (0, 0)),
              pl.BlockSpec((1, HIDDEN), lambda i: (0, 0)),
              pl.BlockSpec((HIDDEN, D_TOT), lambda i: (0, 0)),
              pl.BlockSpec((1, D_TOT), lambda i: (0, 0))]
    out = pl.pallas_call(
        _update_body,
        grid=(_SPLITS[0][1] // _ROWS_BLK,),
        in_specs=[pl.BlockSpec((_ROWS_BLK, D_TOT), lambda i: (i, 0)),
                  pl.BlockSpec((_ROWS_BLK, 2 * HIDDEN), lambda i: (i, 0))]
                 + wspecs,
        out_specs=pl.BlockSpec((_ROWS_BLK, D_TOT), lambda i: (i, 0)),
        out_shape=jax.ShapeDtypeStruct((N, D_TOT), jnp.float32),
    )(x, s_parts[0], w0, b1r, w2p, b2p)

    for (off, nrows), s in zip(_SPLITS[1:], s_parts[1:]):
        blk0 = off // _ROWS_BLK
        out = pl.pallas_call(
            _update_body_hi,
            grid=(nrows // _ROWS_BLK,),
            in_specs=[pl.BlockSpec((8, D_TOT), lambda i: (0, 0)),
                      pl.BlockSpec((_ROWS_BLK, D_TOT),
                                   lambda i, b=blk0: (i + b, 0)),
                      pl.BlockSpec((_ROWS_BLK, 2 * HIDDEN),
                                   lambda i: (i, 0))]
                     + wspecs,
            out_specs=pl.BlockSpec((_ROWS_BLK, D_TOT),
                                   lambda i, b=blk0: (i + b, 0)),
            out_shape=jax.ShapeDtypeStruct((N, D_TOT), jnp.float32),
            input_output_aliases={0: 0},
        )(out, x, s, w0, b1r, w2p, b2p)
    return out
